# Initial kernel scaffold; baseline (speedup 1.0000x reference)
import jax
import jax.numpy as jnp
from jax.experimental import pallas as pl


def kernel(positions, tables):
    raise NotImplementedError("write your pallas kernel here")


# speedup vs baseline: 9.4555x; 9.4555x over previous
import functools

import numpy as np
import jax
import jax.numpy as jnp
from jax import lax
from jax.experimental import pallas as pl
from jax.experimental.pallas import tpu as pltpu
from jax.experimental.pallas import tpu_sc as plsc

N_LEVELS = 16
TABLE_SIZE = 524288
FEATS = 2
BASE_RES = 16
MAX_RES = 2048
_growth = (MAX_RES / BASE_RES) ** (1.0 / (N_LEVELS - 1))
RES_LIST = [int(BASE_RES * _growth**l) for l in range(N_LEVELS)]
PRIME_Y = np.uint32(2654435761).astype(np.int32).item()
PRIME_Z = np.uint32(805459861).astype(np.int32).item()
HASH_MASK = TABLE_SIZE - 1

NUM_WORKERS = 32


def _build_sc_kernel(n_pos: int, chunk: int):
    per_w = n_pos // NUM_WORKERS
    n_chunks = per_w // chunk
    assert per_w % chunk == 0 and chunk % 128 == 0
    j_rows = chunk // 128
    n_groups = chunk // 16
    asm_groups = chunk * FEATS // 16
    out_row = N_LEVELS * FEATS

    mesh = plsc.VectorSubcoreMesh(
        core_axis_name="c", subcore_axis_name="s", num_cores=2, num_subcores=16
    )

    @functools.partial(
        pl.kernel,
        mesh=mesh,
        out_type=jax.ShapeDtypeStruct((n_pos * out_row,), jnp.float32),
        scratch_types=[
            pltpu.VMEM((chunk * 3,), jnp.float32),
            pltpu.VMEM((N_LEVELS * j_rows, 128), jnp.int32),
            pltpu.VMEM((j_rows, 128), jnp.int32),
            pltpu.VMEM((j_rows, 128, 8), jnp.float32),
            pltpu.VMEM((chunk * out_row,), jnp.float32),
            pltpu.SemaphoreType.DMA,
        ],
        compiler_params=pltpu.CompilerParams(
            needs_layout_passes=False, use_tc_tiling_on_sc=False
        ),
    )
    def sc_kernel(pos_hbm, tab_hbm, out_hbm, pos_v, idx_v, sup_v, gath_v, out_v, sem):
        wid = lax.axis_index("s") * 2 + lax.axis_index("c")
        lanes = lax.iota(jnp.int32, 16)
        lane3 = lanes * 3
        half = lanes >> 1
        parity = lanes & 1

        def chunk_body(ci, carry):
            pbase = wid * per_w + ci * chunk
            pltpu.sync_copy(pos_hbm.at[pl.ds(pbase * 3, chunk * 3)], pos_v)

            def hash_body(g, c2):
                off = g * 48
                px = jnp.maximum(plsc.load_gather(pos_v, [off + lane3]), 0.0)
                py = jnp.maximum(plsc.load_gather(pos_v, [off + lane3 + 1]), 0.0)
                pz = jnp.maximum(plsc.load_gather(pos_v, [off + lane3 + 2]), 0.0)
                jrow = g >> 3
                col = (g & 7) * 16
                for l in range(N_LEVELS):
                    res = RES_LIST[l]
                    cx = jnp.minimum((px * res).astype(jnp.int32), res - 1)
                    cy = jnp.minimum((py * res).astype(jnp.int32), res - 1)
                    cz = jnp.minimum((pz * res).astype(jnp.int32), res - 1)
                    h = (cx + cy * PRIME_Y + cz * PRIME_Z) & HASH_MASK
                    idx_v[l * j_rows + jrow, pl.ds(col, 16)] = h + l * TABLE_SIZE
                return c2

            lax.fori_loop(0, n_groups, hash_body, 0, unroll=False)

            def lvl_body(l, c2):
                def fire(j, c3):
                    row = l * j_rows + j

                    def sup(i, c4):
                        e16 = idx_v[row, pl.ds(i * 16, 16)]
                        sup_v[j, pl.ds(i * 16, 16)] = e16 >> 2
                        return c4

                    lax.fori_loop(0, 8, sup, 0, unroll=False)
                    pltpu.async_copy(tab_hbm.at[sup_v.at[j]], gath_v.at[j], sem)
                    return c3

                lax.fori_loop(0, j_rows, fire, 0, unroll=False)

                def drain(j, c3):
                    pltpu.make_async_copy(
                        tab_hbm.at[sup_v.at[j]], gath_v.at[j], sem
                    ).wait()
                    return c3

                lax.fori_loop(0, j_rows, drain, 0, unroll=False)

                def asm(q, c3):
                    jrow = q >> 4
                    r0 = (q & 15) * 8
                    row = jnp.full((16,), l * j_rows + jrow, jnp.int32)
                    jsplat = jnp.full((16,), jrow, jnp.int32)
                    e16 = plsc.load_gather(idx_v, [row, r0 + half])
                    col = ((e16 & 3) << 1) | parity
                    x = plsc.load_gather(gath_v, [jsplat, r0 + half, col])
                    pos = jrow * 128 + r0 + half
                    oidx = pos * out_row + (2 * l + parity)
                    plsc.store_scatter(out_v, [oidx], x)
                    return c3

                lax.fori_loop(0, asm_groups, asm, 0, unroll=False)
                return c2

            lax.fori_loop(0, N_LEVELS, lvl_body, 0, unroll=False)
            pltpu.sync_copy(out_v, out_hbm.at[pl.ds(pbase * out_row, chunk * out_row)])
            return carry

        lax.fori_loop(0, n_chunks, chunk_body, 0, unroll=False)

    return sc_kernel


def _encode(positions, tables, chunk: int):
    n_pos = positions.shape[0]
    pos_flat = positions.reshape(-1)
    tab_flat = tables.reshape(N_LEVELS * TABLE_SIZE // 4, 8)
    fn = _build_sc_kernel(n_pos, chunk)
    out_flat = fn(pos_flat, tab_flat)
    return out_flat.reshape(n_pos, N_LEVELS * FEATS)


def kernel(positions, tables):
    return _encode(positions, tables, chunk=2048)


# speedup vs baseline: 9.4566x; 1.0001x over previous
import functools

import numpy as np
import jax
import jax.numpy as jnp
from jax import lax
from jax.experimental import pallas as pl
from jax.experimental.pallas import tpu as pltpu
from jax.experimental.pallas import tpu_sc as plsc

N_LEVELS = 16
TABLE_SIZE = 524288
FEATS = 2
BASE_RES = 16
MAX_RES = 2048
_growth = (MAX_RES / BASE_RES) ** (1.0 / (N_LEVELS - 1))
RES_LIST = [int(BASE_RES * _growth**l) for l in range(N_LEVELS)]
PRIME_Y = np.uint32(2654435761).astype(np.int32).item()
PRIME_Z = np.uint32(805459861).astype(np.int32).item()
HASH_MASK = TABLE_SIZE - 1

NUM_WORKERS = 32


def _build_sc_kernel(n_pos: int, chunk: int):
    per_w = n_pos // NUM_WORKERS
    n_chunks = per_w // chunk
    assert per_w % chunk == 0 and chunk % 128 == 0
    j_rows = chunk // 128
    n_groups = chunk // 16
    asm_groups = chunk * FEATS // 16
    out_row = N_LEVELS * FEATS

    mesh = plsc.VectorSubcoreMesh(
        core_axis_name="c", subcore_axis_name="s", num_cores=2, num_subcores=16
    )

    @functools.partial(
        pl.kernel,
        mesh=mesh,
        out_type=jax.ShapeDtypeStruct((n_pos, out_row), jnp.float32),
        scratch_types=[
            pltpu.VMEM((chunk * 3,), jnp.float32),
            pltpu.VMEM((N_LEVELS * j_rows, 128), jnp.int32),
            pltpu.VMEM((j_rows, 128), jnp.int32),
            pltpu.VMEM((j_rows, 128, 8), jnp.float32),
            pltpu.VMEM((chunk, out_row), jnp.float32),
            pltpu.SemaphoreType.DMA,
        ],
        compiler_params=pltpu.CompilerParams(
            needs_layout_passes=False, use_tc_tiling_on_sc=False
        ),
    )
    def sc_kernel(pos_hbm, tab_hbm, out_hbm, pos_v, idx_v, sup_v, gath_v, out_v, sem):
        wid = lax.axis_index("s") * 2 + lax.axis_index("c")
        lanes = lax.iota(jnp.int32, 16)
        half = lanes >> 1
        parity = lanes & 1
        zero16 = jnp.zeros((16,), jnp.int32)

        def chunk_body(ci, carry):
            pbase = wid * per_w + ci * chunk
            pltpu.sync_copy(pos_hbm.at[pl.ds(pbase * 3, chunk * 3)], pos_v)

            def hash_body(g, c2):
                off = g * 48 + lanes * 3
                px = jnp.maximum(plsc.load_gather(pos_v, [off]), 0.0)
                py = jnp.maximum(plsc.load_gather(pos_v, [off + 1]), 0.0)
                pz = jnp.maximum(plsc.load_gather(pos_v, [off + 2]), 0.0)
                jrow = g >> 3
                col = (g & 7) * 16
                for l in range(N_LEVELS):
                    res = RES_LIST[l]
                    cx = jnp.minimum((px * res).astype(jnp.int32), res - 1)
                    cy = jnp.minimum((py * res).astype(jnp.int32), res - 1)
                    cz = jnp.minimum((pz * res).astype(jnp.int32), res - 1)
                    h = (cx + cy * PRIME_Y + cz * PRIME_Z) & HASH_MASK
                    idx_v[l * j_rows + jrow, pl.ds(col, 16)] = h + l * TABLE_SIZE
                return c2

            lax.fori_loop(0, n_groups, hash_body, 0, unroll=False)

            def lvl_body(l, c2):
                def fire(j, c3):
                    row = l * j_rows + j

                    def sup(i, c4):
                        e16 = idx_v[row, pl.ds(i * 16, 16)]
                        sup_v[j, pl.ds(i * 16, 16)] = e16 >> 2
                        return c4

                    lax.fori_loop(0, 8, sup, 0, unroll=False)
                    pltpu.async_copy(tab_hbm.at[sup_v.at[j]], gath_v.at[j], sem)
                    return c3

                lax.fori_loop(0, j_rows, fire, 0, unroll=False)

                def drain(j, c3):
                    pltpu.make_async_copy(
                        tab_hbm.at[sup_v.at[j]], gath_v.at[j], sem
                    ).wait()
                    return c3

                lax.fori_loop(0, j_rows, drain, 0, unroll=False)

                def asm(q, c3):
                    jrow = q >> 4
                    r0 = (q & 15) * 8
                    row = jnp.full((16,), l * j_rows + jrow, jnp.int32)
                    jsplat = jnp.full((16,), jrow, jnp.int32)
                    e16 = plsc.load_gather(idx_v, [row, r0 + half])
                    col = ((e16 & 3) << 1) | parity
                    x = plsc.load_gather(gath_v, [jsplat, r0 + half, col])
                    pos = jrow * 128 + r0 + half
                    plsc.store_scatter(out_v, [pos, 2 * l + parity], x)
                    return c3

                lax.fori_loop(0, asm_groups, asm, 0, unroll=False)
                return c2

            lax.fori_loop(0, N_LEVELS, lvl_body, 0, unroll=False)
            pltpu.sync_copy(out_v, out_hbm.at[pl.ds(pbase, chunk)])
            return carry

        lax.fori_loop(0, n_chunks, chunk_body, 0, unroll=False)

    return sc_kernel


def kernel(positions, tables):
    n_pos = positions.shape[0]
    pos_flat = positions.reshape(-1)
    tab_flat = tables.reshape(N_LEVELS * TABLE_SIZE // 4, 8)
    fn = _build_sc_kernel(n_pos, chunk=2048)
    return fn(pos_flat, tab_flat)


# speedup vs baseline: 9.7227x; 1.0281x over previous
import functools

import numpy as np
import jax
import jax.numpy as jnp
from jax import lax
from jax.experimental import pallas as pl
from jax.experimental.pallas import tpu as pltpu
from jax.experimental.pallas import tpu_sc as plsc

N_LEVELS = 16
TABLE_SIZE = 524288
FEATS = 2
BASE_RES = 16
MAX_RES = 2048
_growth = (MAX_RES / BASE_RES) ** (1.0 / (N_LEVELS - 1))
RES_LIST = [int(BASE_RES * _growth**l) for l in range(N_LEVELS)]
PRIME_Y = np.uint32(2654435761).astype(np.int32).item()
PRIME_Z = np.uint32(805459861).astype(np.int32).item()
HASH_MASK = TABLE_SIZE - 1

NUM_WORKERS = 32


def _build_sc_kernel(n_pos: int, chunk: int):
    per_w = n_pos // NUM_WORKERS
    n_chunks = per_w // chunk
    assert per_w % chunk == 0 and chunk % 128 == 0
    j_rows = chunk // 128
    n_groups = chunk // 16
    asm_groups = chunk * FEATS // 16
    out_row = N_LEVELS * FEATS

    mesh = plsc.VectorSubcoreMesh(
        core_axis_name="c", subcore_axis_name="s", num_cores=2, num_subcores=16
    )

    @functools.partial(
        pl.kernel,
        mesh=mesh,
        out_type=jax.ShapeDtypeStruct((4, n_pos // 128, 8, 128), jnp.float32),
        scratch_types=[
            pltpu.VMEM((chunk * 3,), jnp.float32),
            pltpu.VMEM((N_LEVELS * j_rows, 128), jnp.int32),
            pltpu.VMEM((j_rows, 128), jnp.int32),
            pltpu.VMEM((j_rows, 128, 8), jnp.float32),
            pltpu.VMEM((4, chunk // 128, 8, 128), jnp.float32),
            pltpu.SemaphoreType.DMA,
        ],
        compiler_params=pltpu.CompilerParams(
            needs_layout_passes=False, use_tc_tiling_on_sc=False
        ),
    )
    def sc_kernel(pos_hbm, tab_hbm, out_hbm, pos_v, idx_v, sup_v, gath_v, out_v, sem):
        wid = lax.axis_index("s") * 2 + lax.axis_index("c")
        lanes = lax.iota(jnp.int32, 16)
        half = lanes >> 1
        parity = lanes & 1
        zero16 = jnp.zeros((16,), jnp.int32)

        def chunk_body(ci, carry):
            pbase = wid * per_w + ci * chunk
            pltpu.sync_copy(pos_hbm.at[pl.ds(pbase * 3, chunk * 3)], pos_v)

            def hash_body(g, c2):
                off = g * 48 + lanes * 3
                px = jnp.maximum(plsc.load_gather(pos_v, [off]), 0.0)
                py = jnp.maximum(plsc.load_gather(pos_v, [off + 1]), 0.0)
                pz = jnp.maximum(plsc.load_gather(pos_v, [off + 2]), 0.0)
                jrow = g >> 3
                col = (g & 7) * 16
                for l in range(N_LEVELS):
                    res = RES_LIST[l]
                    cx = jnp.minimum((px * res).astype(jnp.int32), res - 1)
                    cy = jnp.minimum((py * res).astype(jnp.int32), res - 1)
                    cz = jnp.minimum((pz * res).astype(jnp.int32), res - 1)
                    h = (cx + cy * PRIME_Y + cz * PRIME_Z) & HASH_MASK
                    idx_v[l * j_rows + jrow, pl.ds(col, 16)] = h + l * TABLE_SIZE
                return c2

            lax.fori_loop(0, n_groups, hash_body, 0, unroll=False)

            def lvl_body(l, c2):
                def fire(j, c3):
                    row = l * j_rows + j

                    def sup(i, c4):
                        e16 = idx_v[row, pl.ds(i * 16, 16)]
                        sup_v[j, pl.ds(i * 16, 16)] = e16 >> 2
                        return c4

                    lax.fori_loop(0, 8, sup, 0, unroll=False)
                    pltpu.async_copy(tab_hbm.at[sup_v.at[j]], gath_v.at[j], sem)
                    return c3

                lax.fori_loop(0, j_rows, fire, 0, unroll=False)

                def drain(j, c3):
                    pltpu.make_async_copy(
                        tab_hbm.at[sup_v.at[j]], gath_v.at[j], sem
                    ).wait()
                    return c3

                lax.fori_loop(0, j_rows, drain, 0, unroll=False)

                def asm(q, c3):
                    jrow = q >> 4
                    r0 = (q & 15) * 8
                    row = jnp.full((16,), l * j_rows + jrow, jnp.int32)
                    jsplat = jnp.full((16,), jrow, jnp.int32)
                    e16 = plsc.load_gather(idx_v, [row, r0 + half])
                    col = ((e16 & 3) << 1) | parity
                    x = plsc.load_gather(gath_v, [jsplat, r0 + half, col])
                    c = 2 * l + parity
                    plsc.store_scatter(
                        out_v, [c >> 3, jsplat, c & 7, r0 + half], x
                    )
                    return c3

                lax.fori_loop(0, asm_groups, asm, 0, unroll=False)
                return c2

            lax.fori_loop(0, N_LEVELS, lvl_body, 0, unroll=False)
            pblk = pbase // 128
            for fb in range(4):
                pltpu.sync_copy(
                    out_v.at[fb], out_hbm.at[fb, pl.ds(pblk, chunk // 128)]
                )
            return carry

        lax.fori_loop(0, n_chunks, chunk_body, 0, unroll=False)

    return sc_kernel


def kernel(positions, tables):
    n_pos = positions.shape[0]
    pos_flat = positions.reshape(-1)
    tab_flat = tables.reshape(N_LEVELS * TABLE_SIZE // 4, 8)
    fn = _build_sc_kernel(n_pos, chunk=2048)
    out4 = fn(pos_flat, tab_flat)
    return out4.transpose(1, 3, 0, 2).reshape(n_pos, N_LEVELS * FEATS)


# speedup vs baseline: 48.2199x; 4.9595x over previous
import functools

import numpy as np
import jax
import jax.numpy as jnp
from jax import lax
from jax.experimental import pallas as pl
from jax.experimental.pallas import tpu as pltpu
from jax.experimental.pallas import tpu_sc as plsc

N_LEVELS = 16
TABLE_SIZE = 524288
FEATS = 2
BASE_RES = 16
MAX_RES = 2048
_growth = (MAX_RES / BASE_RES) ** (1.0 / (N_LEVELS - 1))
RES_LIST = [int(BASE_RES * _growth**l) for l in range(N_LEVELS)]
PRIME_Y = np.uint32(2654435761).astype(np.int32).item()
PRIME_Z = np.uint32(805459861).astype(np.int32).item()
HASH_MASK = TABLE_SIZE - 1
LEVEL_FLOATS = TABLE_SIZE * FEATS

NUM_WORKERS = 32


def _build_sc_kernel(n_pos: int, chunk: int):
    per_w = n_pos // NUM_WORKERS
    n_chunks = per_w // chunk
    assert per_w % chunk == 0 and chunk % 128 == 0
    j_rows = chunk // 128
    n_groups = chunk // 16
    asm_groups = chunk * FEATS // 16

    mesh = plsc.VectorSubcoreMesh(
        core_axis_name="c", subcore_axis_name="s", num_cores=2, num_subcores=16
    )

    @functools.partial(
        pl.kernel,
        mesh=mesh,
        out_type=jax.ShapeDtypeStruct((4, n_pos // 128, 8, 128), jnp.float32),
        scratch_types=[
            pltpu.VMEM((chunk,), jnp.float32),
            pltpu.VMEM((chunk,), jnp.float32),
            pltpu.VMEM((chunk,), jnp.float32),
            pltpu.VMEM((N_LEVELS * j_rows, 128), jnp.int32),
            pltpu.VMEM((j_rows, 128), jnp.int32),
            pltpu.VMEM((j_rows, 128), jnp.int32),
            pltpu.VMEM((2, j_rows, 128, 8), jnp.float32),
            pltpu.VMEM((4, chunk // 128, 8, 128), jnp.float32),
            pltpu.SemaphoreType.DMA,
        ],
        compiler_params=pltpu.CompilerParams(
            needs_layout_passes=False, use_tc_tiling_on_sc=False
        ),
    )
    def sc_kernel(px_hbm, py_hbm, pz_hbm, tab_hbm, out_hbm,
                  px_v, py_v, pz_v, idx_v, sup0_v, sup1_v, gath_v, out_v, sem):
        wid = lax.axis_index("s") * 2 + lax.axis_index("c")
        lanes = lax.iota(jnp.int32, 16)
        half = lanes >> 1
        parity = lanes & 1

        def chunk_body(ci, carry):
            pbase = wid * per_w + ci * chunk
            pltpu.sync_copy(px_hbm.at[pl.ds(pbase, chunk)], px_v)
            pltpu.sync_copy(py_hbm.at[pl.ds(pbase, chunk)], py_v)
            pltpu.sync_copy(pz_hbm.at[pl.ds(pbase, chunk)], pz_v)

            def hash_body(g, c2):
                sl = pl.ds(g * 16, 16)
                px = jnp.maximum(px_v[sl], 0.0)
                py = jnp.maximum(py_v[sl], 0.0)
                pz = jnp.maximum(pz_v[sl], 0.0)
                jrow = g >> 3
                col = (g & 7) * 16
                for l in range(N_LEVELS):
                    res = RES_LIST[l]
                    cx = jnp.minimum((px * res).astype(jnp.int32), res - 1)
                    cy = jnp.minimum((py * res).astype(jnp.int32), res - 1)
                    cz = jnp.minimum((pz * res).astype(jnp.int32), res - 1)
                    h = (cx + cy * PRIME_Y + cz * PRIME_Z) & HASH_MASK
                    flat = ((h >> 7) << 8) | (h & 127)
                    idx_v[l * j_rows + jrow, pl.ds(col, 16)] = flat + l * LEVEL_FLOATS
                return c2

            lax.fori_loop(0, n_groups, hash_body, 0, unroll=False)

            def lvl_body(l, c2):
                def fire(j, c3):
                    row = l * j_rows + j

                    def sup(i, c4):
                        f16 = idx_v[row, pl.ds(i * 16, 16)]
                        s0 = f16 >> 3
                        sup0_v[j, pl.ds(i * 16, 16)] = s0
                        sup1_v[j, pl.ds(i * 16, 16)] = s0 + 16
                        return c4

                    lax.fori_loop(0, 8, sup, 0, unroll=False)
                    pltpu.async_copy(tab_hbm.at[sup0_v.at[j]], gath_v.at[0, j], sem)
                    pltpu.async_copy(tab_hbm.at[sup1_v.at[j]], gath_v.at[1, j], sem)
                    return c3

                lax.fori_loop(0, j_rows, fire, 0, unroll=False)

                def drain(j, c3):
                    pltpu.make_async_copy(
                        tab_hbm.at[sup0_v.at[j]], gath_v.at[0, j], sem
                    ).wait()
                    pltpu.make_async_copy(
                        tab_hbm.at[sup1_v.at[j]], gath_v.at[1, j], sem
                    ).wait()
                    return c3

                lax.fori_loop(0, j_rows, drain, 0, unroll=False)

                def asm(q, c3):
                    jrow = q >> 4
                    r0 = (q & 15) * 8
                    row = jnp.full((16,), l * j_rows + jrow, jnp.int32)
                    jsplat = jnp.full((16,), jrow, jnp.int32)
                    f16 = plsc.load_gather(idx_v, [row, r0 + half])
                    sub = f16 & 7
                    x = plsc.load_gather(gath_v, [parity, jsplat, r0 + half, sub])
                    c = 2 * l + parity
                    plsc.store_scatter(
                        out_v, [c >> 3, jsplat, c & 7, r0 + half], x
                    )
                    return c3

                lax.fori_loop(0, asm_groups, asm, 0, unroll=False)
                return c2

            lax.fori_loop(0, N_LEVELS, lvl_body, 0, unroll=False)
            pblk = pbase // 128
            for fb in range(4):
                pltpu.sync_copy(
                    out_v.at[fb], out_hbm.at[fb, pl.ds(pblk, chunk // 128)]
                )
            return carry

        lax.fori_loop(0, n_chunks, chunk_body, 0, unroll=False)

    return sc_kernel


def kernel(positions, tables):
    n_pos = positions.shape[0]
    px = positions[:, 0]
    py = positions[:, 1]
    pz = positions[:, 2]
    tab_flat = (tables.reshape(N_LEVELS, 4096, 128, 2)
                .swapaxes(2, 3).reshape(N_LEVELS * TABLE_SIZE // 4, 8))
    fn = _build_sc_kernel(n_pos, chunk=1024)
    out4 = fn(px, py, pz, tab_flat)
    return out4.transpose(1, 3, 0, 2).reshape(n_pos, N_LEVELS * FEATS)
